# Initial kernel scaffold; baseline (speedup 1.0000x reference)
import jax
import jax.numpy as jnp
from jax.experimental import pallas as pl


def kernel(continuous, categorical_indices, tables, W0, b0, W1, b1, W2, b2):
    raise NotImplementedError("write your pallas kernel here")


# speedup vs baseline: 12.8919x; 12.8919x over previous
import jax
import jax.numpy as jnp
from jax.experimental import pallas as pl
from jax.experimental.pallas import tpu as pltpu
from jax.experimental.pallas import tpu_sc as plsc

B = 16384
F = 26
V = 100000
D = 32
C = 16

GATHER_WINDOW = 128
MLP_BLOCK = 1024


def _sc_gather(tables, flat_idx):
    n = flat_idx.shape[1]
    mesh = plsc.VectorSubcoreMesh(core_axis_name="core", subcore_axis_name="subcore")

    @pl.kernel(out_type=jax.ShapeDtypeStruct((n, D), tables.dtype), mesh=mesh)
    def gather_kernel(tab_hbm, idx_hbm, out_hbm):
        def body(idx_vmem, out_vmem):
            pltpu.sync_copy(tab_hbm.at[idx_vmem.at[0]], out_vmem)

        pltpu.emit_pipeline(
            body,
            grid=(n // GATHER_WINDOW,),
            in_specs=[pl.BlockSpec((1, GATHER_WINDOW), index_map=lambda i: (0, i))],
            out_specs=[pl.BlockSpec((GATHER_WINDOW, D), index_map=lambda i: (i, 0))],
            core_axis_name=("core", "subcore"),
            dimension_semantics=(pltpu.PARALLEL,),
        )(idx_hbm, out_hbm)

    return gather_kernel(tables, flat_idx)


def _mlp_kernel(emb_ref, cont_ref, w0e_ref, w0c_ref, b0_ref, w1_ref, b1_ref,
                w2_ref, b2_ref, out_ref):
    x = jnp.dot(emb_ref[...], w0e_ref[...], preferred_element_type=jnp.float32)
    x = x + jnp.dot(cont_ref[...], w0c_ref[...], preferred_element_type=jnp.float32)
    x = jnp.maximum(x + b0_ref[...], 0.0)
    x = jnp.maximum(jnp.dot(x, w1_ref[...], preferred_element_type=jnp.float32)
                    + b1_ref[...], 0.0)
    x = jnp.maximum(jnp.dot(x, w2_ref[...], preferred_element_type=jnp.float32)
                    + b2_ref[...], 0.0)
    out_ref[...] = x


def _mlp(emb, cont, W0e, W0c, b0, W1, b1, W2, b2):
    grid = (B // MLP_BLOCK,)
    full = lambda shape: pl.BlockSpec(shape, lambda i: (0, 0))
    return pl.pallas_call(
        _mlp_kernel,
        grid=grid,
        in_specs=[
            pl.BlockSpec((MLP_BLOCK, F * D), lambda i: (i, 0)),
            pl.BlockSpec((MLP_BLOCK, C), lambda i: (i, 0)),
            full(W0e.shape), full(W0c.shape), full(b0.shape),
            full(W1.shape), full(b1.shape), full(W2.shape), full(b2.shape),
        ],
        out_specs=pl.BlockSpec((MLP_BLOCK, W2.shape[1]), lambda i: (i, 0)),
        out_shape=jax.ShapeDtypeStruct((B, W2.shape[1]), jnp.float32),
    )(emb, cont, W0e, W0c, b0, W1, b1, W2, b2)


def kernel(continuous, categorical_indices, tables, W0, b0, W1, b1, W2, b2):
    offsets = (jnp.arange(F, dtype=categorical_indices.dtype) * V)[None, :]
    flat_idx = (categorical_indices + offsets).reshape(-1)
    emb = jnp.take(tables, flat_idx, axis=0).reshape(B, F * D)
    W0e = W0[: F * D]
    W0c = W0[F * D:]
    return _mlp(emb, continuous, W0e, W0c, b0[None, :], W1, b1[None, :],
                W2, b2[None, :])
